# baseline (speedup 1.0000x reference)
import functools

import jax
import jax.numpy as jnp
from jax.experimental import pallas as pl
from jax.experimental.pallas import tpu as pltpu
from jax.experimental.pallas import tpu_sc as plsc

N = 50000
NUM_GRAPHS = 256
H = 128
R = 512
NB = 98
NP = NB * R
NP2 = NP + R
C = 144
GW = 128
PC = 19
OFFS = (-4, -3, -2, -1, 1, 2, 3, 4)
NEG = -1e30
KL = N - 1 - ((NB - 1) * R - 4)


def _gather_rows(src, idx):
    rows, cols = idx.shape[0], src.shape[1]
    steps = rows // GW
    idx2 = idx.reshape(1, rows)
    mesh = plsc.VectorSubcoreMesh(core_axis_name="c", subcore_axis_name="s")

    @functools.partial(
        pl.kernel,
        out_type=jax.ShapeDtypeStruct((rows, cols), src.dtype),
        mesh=mesh,
    )
    def gk(x_hbm, i_hbm, o_hbm):
        def body(i_vmem, o_vmem):
            pltpu.sync_copy(x_hbm.at[i_vmem.at[0]], o_vmem)

        pltpu.emit_pipeline(
            body,
            grid=(steps,),
            in_specs=[pl.BlockSpec((1, GW), lambda i: (0, i))],
            out_specs=[pl.BlockSpec((GW, cols), lambda i: (i, 0))],
            core_axis_name=("c", "s"),
            dimension_semantics=(pltpu.PARALLEL,),
        )(i_hbm, o_hbm)

    return gk(src, idx2)


def _glu_body(xg, wlf, wgf, vlin, vgate, bl, bg, hout):
    feat = xg[:, 1:129]
    tc = xg[:, 0:1]
    lin = jnp.dot(feat, wlf[...], preferred_element_type=jnp.float32) \
        + tc * vlin[...] + bl[...]
    gate = jnp.dot(feat, wgf[...], preferred_element_type=jnp.float32) \
        + tc * vgate[...] + bg[...]
    hout[...] = lin * jax.nn.sigmoid(gate)


def _main_body(glohi, hs_p, hs_c, hs_n, pp, pc, pn, bs_c,
               w1a, w1b, b1, w2a, w2b, b2,
               omax, osum, ocnt):
    b = pl.program_id(0)
    f32 = jnp.float32

    @pl.when(b == 0)
    def _init():
        omax[...] = jnp.full_like(omax, -jnp.inf)
        osum[...] = jnp.zeros_like(osum)
        ocnt[...] = jnp.zeros_like(ocnt)

    h16 = jnp.concatenate([hs_p[R - 8:], hs_c[...], hs_n[:8]], axis=0)
    pst = jnp.concatenate([pp[R - 8:], pc[...], pn[:8]], axis=0)

    M = R + 16
    A1 = jnp.dot(h16[4:M - 4], w1a[...], preferred_element_type=f32) + b1[...]
    B1 = jnp.dot(h16, w1b[...], preferred_element_type=f32)
    p1 = pst[4:M - 4]
    T = None
    for j, d in enumerate(OFFS):
        cand = B1[4 + d:M - 4 + d] + p1[:, j:j + 1]
        T = cand if T is None else jnp.maximum(T, cand)
    x1 = jax.nn.relu(
        A1 + jnp.maximum(T - B1[4:M - 4], p1[:, 18:19]))

    M2 = R + 8
    A2 = jnp.dot(x1[4:M2 - 4], w2a[...], preferred_element_type=f32) + b2[...]
    B2 = jnp.dot(x1, w2b[...], preferred_element_type=f32)
    p2 = pst[8:R + 8]
    T2 = None
    for j, d in enumerate(OFFS):
        cand = B2[4 + d:M2 - 4 + d] + p2[:, 8 + j:9 + j]
        T2 = cand if T2 is None else jnp.maximum(T2, cand)
    T2 = jnp.maximum(T2, B2[4:5] + p2[:, 16:17])
    T2 = jnp.maximum(T2, B2[KL:KL + 1] + p2[:, 17:18])
    x2 = jax.nn.relu(
        A2 + jnp.maximum(T2 - B2[4:M2 - 4], p2[:, 18:19]))

    comb = jnp.concatenate([x1[4:R + 4], x2], axis=1)

    s = b * R
    bsc = bs_c[...]
    growc = s + jax.lax.broadcasted_iota(jnp.int32, (R, 1), 0)
    rowok = growc < N
    glo = glohi[0, b]
    ghi = glohi[1, b]

    def body(gi, carry):
        m = (bsc == gi) & rowok
        mx = jnp.max(jnp.where(m, comb, -jnp.inf), axis=0, keepdims=True)
        sm = jnp.sum(jnp.where(m, comb, 0.0), axis=0, keepdims=True)
        cn = jnp.sum(m.astype(f32), keepdims=True)
        omax[pl.ds(gi, 1), :] = jnp.maximum(omax[pl.ds(gi, 1), :], mx)
        osum[pl.ds(gi, 1), :] = osum[pl.ds(gi, 1), :] + sm
        ocnt[pl.ds(gi, 1), :] = ocnt[pl.ds(gi, 1), :] + cn
        return carry

    jax.lax.fori_loop(glo, ghi + 1, body, 0)


def _head_body(pmax, psum, cnt, wfl, wfg, bfl, bfg, wo, bo, out):
    c = cnt[...]
    maxp = jnp.where(c > 0, pmax[...], 0.0)
    meanp = psum[...] / jnp.maximum(c, 1.0)
    pooled = jnp.concatenate([maxp, meanp], axis=1)
    lin = jnp.dot(pooled, wfl[...], preferred_element_type=jnp.float32) + bfl[...]
    gate = jnp.dot(pooled, wfg[...], preferred_element_type=jnp.float32) + bfg[...]
    hh = lin * jax.nn.sigmoid(gate)
    logits = jnp.dot(hh, wo[...], preferred_element_type=jnp.float32) + bo[...]
    lanes = jax.lax.broadcasted_iota(jnp.int32, logits.shape, 1)
    ok = lanes < 2
    m = jnp.max(jnp.where(ok, logits, -jnp.inf), axis=1, keepdims=True)
    e = jnp.where(ok, jnp.exp(logits - m), 0.0)
    ls = logits - m - jnp.log(jnp.sum(e, axis=1, keepdims=True))
    out[...] = ls[:, 0:2]


def _penalties(batch):
    rows2 = jnp.arange(NP2, dtype=jnp.int32)[:, None]
    inreal = (rows2 >= R) & (rows2 < R + N)
    g = jnp.clip(rows2 - R, 0, N - 1)
    d = jnp.asarray(OFFS, jnp.int32)[None, :]
    idx = jnp.clip(g + d, 0, N - 1)
    bg = batch[g[:, 0]][:, None]
    vclip = (idx != g) & (batch[idx] == bg)
    inr = (g + d >= 0) & (g + d <= N - 1)
    pen1 = jnp.where(vclip & inreal, 0.0, NEG)
    pen2 = jnp.where(vclip & inr & inreal, 0.0, NEG)
    x0 = (g >= 1) & (g <= 3) & (bg == batch[0]) & inreal
    xn = (g >= N - 4) & (g != N - 1) & (bg == batch[N - 1]) & inreal
    floor = jnp.where(jnp.all(vclip, axis=1, keepdims=True) & inreal, NEG, 0.0)
    return jnp.concatenate(
        [pen1, pen2, jnp.where(x0, 0.0, NEG), jnp.where(xn, 0.0, NEG), floor],
        axis=1).astype(jnp.float32)


def kernel(x, batch, Wt, bt, Wl, bl, Wg, bg, W1, b1, W2, b2,
           Wfl, bfl, Wfg, bfg, Wo, bo):
    f32 = jnp.float32
    t = x[:, 0]
    xp = jnp.pad(x, ((0, NP - N), (0, 0)))

    order = jnp.lexsort((t, batch)).astype(jnp.int32)
    order2 = jnp.concatenate([
        jnp.full((R,), order[0], jnp.int32),
        order,
        jnp.full((NP - N,), order[N - 1], jnp.int32),
    ])

    pens = _penalties(batch)

    batchp2 = jnp.pad(batch, (R, NP - N), mode="edge").reshape(NP2, 1)
    blo = batch[jnp.arange(NB, dtype=jnp.int32) * R]
    bhi = batch[jnp.minimum((jnp.arange(NB, dtype=jnp.int32) + 1) * R, N) - 1]
    glohi = jnp.stack([blo, bhi]).astype(jnp.int32)

    wlf = Wl[:, :128].T
    wgf = Wg[:, :128].T
    vlin = (Wl[:, 128:] @ Wt[:, 0]).reshape(1, H)
    vgate = (Wg[:, 128:] @ Wt[:, 0]).reshape(1, H)
    bl_e = (bl + Wl[:, 128:] @ bt).reshape(1, H)
    bg_e = (bg + Wg[:, 128:] @ bt).reshape(1, H)
    w1a, w1b = W1[:, :128].T, W1[:, 128:].T
    w2a, w2b = W2[:, :128].T, W2[:, 128:].T
    b1_, b2_ = b1.reshape(1, H), b2.reshape(1, H)

    csimple = lambda shape: pl.BlockSpec(shape, lambda b: (0, 0))
    h = pl.pallas_call(
        _glu_body,
        grid=(NB,),
        in_specs=[
            pl.BlockSpec((R, 129), lambda b: (b, 0)),
            csimple((H, H)), csimple((H, H)),
            csimple((1, H)), csimple((1, H)),
            csimple((1, H)), csimple((1, H)),
        ],
        out_specs=pl.BlockSpec((R, H), lambda b: (b, 0)),
        out_shape=jax.ShapeDtypeStruct((NP, H), f32),
    )(xp, wlf, wgf, vlin, vgate, bl_e, bg_e)

    hs = _gather_rows(h, order2)

    const_spec = lambda shape: pl.BlockSpec(shape, lambda b, g: (0, 0))
    prev_map = lambda b, g: (b, 0)
    cent_map = lambda b, g: (b + 1, 0)
    next_map = lambda b, g: (jnp.minimum(b + 2, NB), 0)

    grid_spec = pltpu.PrefetchScalarGridSpec(
        num_scalar_prefetch=1,
        grid=(NB,),
        in_specs=[
            pl.BlockSpec((R, H), prev_map),
            pl.BlockSpec((R, H), cent_map),
            pl.BlockSpec((R, H), next_map),
            pl.BlockSpec((R, PC), prev_map),
            pl.BlockSpec((R, PC), cent_map),
            pl.BlockSpec((R, PC), next_map),
            pl.BlockSpec((R, 1), cent_map),
            const_spec((H, H)), const_spec((H, H)), const_spec((1, H)),
            const_spec((H, H)), const_spec((H, H)), const_spec((1, H)),
        ],
        out_specs=[
            pl.BlockSpec((NUM_GRAPHS, 2 * H), lambda b, g: (0, 0)),
            pl.BlockSpec((NUM_GRAPHS, 2 * H), lambda b, g: (0, 0)),
            pl.BlockSpec((NUM_GRAPHS, 1), lambda b, g: (0, 0)),
        ],
    )
    pmax, psum, cnt = pl.pallas_call(
        _main_body,
        grid_spec=grid_spec,
        out_shape=[
            jax.ShapeDtypeStruct((NUM_GRAPHS, 2 * H), f32),
            jax.ShapeDtypeStruct((NUM_GRAPHS, 2 * H), f32),
            jax.ShapeDtypeStruct((NUM_GRAPHS, 1), f32),
        ],
    )(glohi, hs, hs, hs, pens, pens, pens, batchp2,
      w1a, w1b, b1_, w2a, w2b, b2_)

    wo128 = jnp.pad(Wo.T, ((0, 0), (0, H - 2)))
    bo128 = jnp.pad(bo.reshape(1, 2), ((0, 0), (0, H - 2)))
    out = pl.pallas_call(
        _head_body,
        out_shape=jax.ShapeDtypeStruct((NUM_GRAPHS, 2), f32),
    )(pmax, psum, cnt, Wfl.T, Wfg.T, bfl.reshape(1, H), bfg.reshape(1, H),
      wo128, bo128)
    return out


# speedup vs baseline: 1.0266x; 1.0266x over previous
import functools

import jax
import jax.numpy as jnp
from jax.experimental import pallas as pl
from jax.experimental.pallas import tpu as pltpu
from jax.experimental.pallas import tpu_sc as plsc

N = 50000
NUM_GRAPHS = 256
H = 128
R = 512
NB = 98
NP = NB * R
NP2 = NP + R
C = 144
GW = 128
PC = 19
OFFS = (-4, -3, -2, -1, 1, 2, 3, 4)
NEG = -1e30
KL = N - 1 - ((NB - 1) * R - 4)


def _gather_rows(src, idx):
    rows, cols = idx.shape[0], src.shape[1]
    steps = rows // GW
    idx2 = idx.reshape(1, rows)
    mesh = plsc.VectorSubcoreMesh(core_axis_name="c", subcore_axis_name="s")

    @functools.partial(
        pl.kernel,
        out_type=jax.ShapeDtypeStruct((rows, cols), src.dtype),
        mesh=mesh,
    )
    def gk(x_hbm, i_hbm, o_hbm):
        def body(i_vmem, o_vmem):
            pltpu.sync_copy(x_hbm.at[i_vmem.at[0]], o_vmem)

        pltpu.emit_pipeline(
            body,
            grid=(steps,),
            in_specs=[pl.BlockSpec((1, GW), lambda i: (0, i))],
            out_specs=[pl.BlockSpec((GW, cols), lambda i: (i, 0))],
            core_axis_name=("c", "s"),
            dimension_semantics=(pltpu.PARALLEL,),
        )(i_hbm, o_hbm)

    return gk(src, idx2)


def _glu_body(xg, wlf, wgf, vlin, vgate, bl, bg, hout):
    feat = xg[:, :128]
    tc = xg[:, 128:129]
    lin = jnp.dot(feat, wlf[...], preferred_element_type=jnp.float32) \
        + tc * vlin[...] + bl[...]
    gate = jnp.dot(feat, wgf[...], preferred_element_type=jnp.float32) \
        + tc * vgate[...] + bg[...]
    hout[...] = lin * jax.nn.sigmoid(gate)


def _main_body(glohi, hs_p, hs_c, hs_n, pp, pc, pn, bs_c,
               w1a, w1b, b1, w2a, w2b, b2,
               omax, osum, ocnt):
    b = pl.program_id(0)
    f32 = jnp.float32

    @pl.when(b == 0)
    def _init():
        omax[...] = jnp.full_like(omax, -jnp.inf)
        osum[...] = jnp.zeros_like(osum)
        ocnt[...] = jnp.zeros_like(ocnt)

    h16 = jnp.concatenate([hs_p[R - 8:], hs_c[...], hs_n[:8]], axis=0)
    pst = jnp.concatenate([pp[R - 8:], pc[...], pn[:8]], axis=0)

    M = R + 16
    A1 = jnp.dot(h16[4:M - 4], w1a[...], preferred_element_type=f32) + b1[...]
    B1 = jnp.dot(h16, w1b[...], preferred_element_type=f32)
    p1 = pst[4:M - 4]
    T = None
    for j, d in enumerate(OFFS):
        cand = B1[4 + d:M - 4 + d] + p1[:, j:j + 1]
        T = cand if T is None else jnp.maximum(T, cand)
    x1 = jax.nn.relu(
        A1 + jnp.maximum(T - B1[4:M - 4], p1[:, 18:19]))

    M2 = R + 8
    A2 = jnp.dot(x1[4:M2 - 4], w2a[...], preferred_element_type=f32) + b2[...]
    B2 = jnp.dot(x1, w2b[...], preferred_element_type=f32)
    p2 = pst[8:R + 8]
    T2 = None
    for j, d in enumerate(OFFS):
        cand = B2[4 + d:M2 - 4 + d] + p2[:, 8 + j:9 + j]
        T2 = cand if T2 is None else jnp.maximum(T2, cand)
    T2 = jnp.maximum(T2, B2[4:5] + p2[:, 16:17])
    T2 = jnp.maximum(T2, B2[KL:KL + 1] + p2[:, 17:18])
    x2 = jax.nn.relu(
        A2 + jnp.maximum(T2 - B2[4:M2 - 4], p2[:, 18:19]))

    comb = jnp.concatenate([x1[4:R + 4], x2], axis=1)

    s = b * R
    bsc = bs_c[...]
    growc = s + jax.lax.broadcasted_iota(jnp.int32, (R, 1), 0)
    rowok = growc < N
    glo = glohi[0, b]
    ghi = glohi[1, b]

    def body(gi, carry):
        m = (bsc == gi) & rowok
        mx = jnp.max(jnp.where(m, comb, -jnp.inf), axis=0, keepdims=True)
        sm = jnp.sum(jnp.where(m, comb, 0.0), axis=0, keepdims=True)
        cn = jnp.sum(m.astype(f32), keepdims=True)
        omax[pl.ds(gi, 1), :] = jnp.maximum(omax[pl.ds(gi, 1), :], mx)
        osum[pl.ds(gi, 1), :] = osum[pl.ds(gi, 1), :] + sm
        ocnt[pl.ds(gi, 1), :] = ocnt[pl.ds(gi, 1), :] + cn
        return carry

    jax.lax.fori_loop(glo, ghi + 1, body, 0)


def _head_body(pmax, psum, cnt, wfl, wfg, bfl, bfg, wo, bo, out):
    c = cnt[...]
    maxp = jnp.where(c > 0, pmax[...], 0.0)
    meanp = psum[...] / jnp.maximum(c, 1.0)
    pooled = jnp.concatenate([maxp, meanp], axis=1)
    lin = jnp.dot(pooled, wfl[...], preferred_element_type=jnp.float32) + bfl[...]
    gate = jnp.dot(pooled, wfg[...], preferred_element_type=jnp.float32) + bfg[...]
    hh = lin * jax.nn.sigmoid(gate)
    logits = jnp.dot(hh, wo[...], preferred_element_type=jnp.float32) + bo[...]
    lanes = jax.lax.broadcasted_iota(jnp.int32, logits.shape, 1)
    ok = lanes < 2
    m = jnp.max(jnp.where(ok, logits, -jnp.inf), axis=1, keepdims=True)
    e = jnp.where(ok, jnp.exp(logits - m), 0.0)
    ls = logits - m - jnp.log(jnp.sum(e, axis=1, keepdims=True))
    out[...] = ls[:, 0:2]


def _penalties(batch):
    rows2 = jnp.arange(NP2, dtype=jnp.int32)[:, None]
    inreal = (rows2 >= R) & (rows2 < R + N)
    g = jnp.clip(rows2 - R, 0, N - 1)
    d = jnp.asarray(OFFS, jnp.int32)[None, :]
    idx = jnp.clip(g + d, 0, N - 1)
    bg = batch[g[:, 0]][:, None]
    vclip = (idx != g) & (batch[idx] == bg)
    inr = (g + d >= 0) & (g + d <= N - 1)
    pen1 = jnp.where(vclip & inreal, 0.0, NEG)
    pen2 = jnp.where(vclip & inr & inreal, 0.0, NEG)
    x0 = (g >= 1) & (g <= 3) & (bg == batch[0]) & inreal
    xn = (g >= N - 4) & (g != N - 1) & (bg == batch[N - 1]) & inreal
    floor = jnp.where(jnp.all(vclip, axis=1, keepdims=True) & inreal, NEG, 0.0)
    return jnp.concatenate(
        [pen1, pen2, jnp.where(x0, 0.0, NEG), jnp.where(xn, 0.0, NEG), floor],
        axis=1).astype(jnp.float32)


def kernel(x, batch, Wt, bt, Wl, bl, Wg, bg, W1, b1, W2, b2,
           Wfl, bfl, Wfg, bfg, Wo, bo):
    f32 = jnp.float32
    t = x[:, 0]
    xr = jnp.concatenate([x[:, 1:], x[:, :1]], axis=1)
    xp = jnp.pad(xr, ((0, NP - N), (0, C - x.shape[1])))

    order = jnp.lexsort((t, batch)).astype(jnp.int32)
    order2 = jnp.concatenate([
        jnp.full((R,), order[0], jnp.int32),
        order,
        jnp.full((NP - N,), order[N - 1], jnp.int32),
    ])

    pens = _penalties(batch)

    batchp2 = jnp.pad(batch, (R, NP - N), mode="edge").reshape(NP2, 1)
    blo = batch[jnp.arange(NB, dtype=jnp.int32) * R]
    bhi = batch[jnp.minimum((jnp.arange(NB, dtype=jnp.int32) + 1) * R, N) - 1]
    glohi = jnp.stack([blo, bhi]).astype(jnp.int32)

    wlf = Wl[:, :128].T
    wgf = Wg[:, :128].T
    vlin = (Wl[:, 128:] @ Wt[:, 0]).reshape(1, H)
    vgate = (Wg[:, 128:] @ Wt[:, 0]).reshape(1, H)
    bl_e = (bl + Wl[:, 128:] @ bt).reshape(1, H)
    bg_e = (bg + Wg[:, 128:] @ bt).reshape(1, H)
    w1a, w1b = W1[:, :128].T, W1[:, 128:].T
    w2a, w2b = W2[:, :128].T, W2[:, 128:].T
    b1_, b2_ = b1.reshape(1, H), b2.reshape(1, H)

    csimple = lambda shape: pl.BlockSpec(shape, lambda b: (0, 0))
    h = pl.pallas_call(
        _glu_body,
        grid=(NB,),
        in_specs=[
            pl.BlockSpec((R, C), lambda b: (b, 0)),
            csimple((H, H)), csimple((H, H)),
            csimple((1, H)), csimple((1, H)),
            csimple((1, H)), csimple((1, H)),
        ],
        out_specs=pl.BlockSpec((R, H), lambda b: (b, 0)),
        out_shape=jax.ShapeDtypeStruct((NP, H), f32),
    )(xp, wlf, wgf, vlin, vgate, bl_e, bg_e)

    hs = _gather_rows(h, order2)

    const_spec = lambda shape: pl.BlockSpec(shape, lambda b, g: (0, 0))
    prev_map = lambda b, g: (b, 0)
    cent_map = lambda b, g: (b + 1, 0)
    next_map = lambda b, g: (jnp.minimum(b + 2, NB), 0)

    grid_spec = pltpu.PrefetchScalarGridSpec(
        num_scalar_prefetch=1,
        grid=(NB,),
        in_specs=[
            pl.BlockSpec((R, H), prev_map),
            pl.BlockSpec((R, H), cent_map),
            pl.BlockSpec((R, H), next_map),
            pl.BlockSpec((R, PC), prev_map),
            pl.BlockSpec((R, PC), cent_map),
            pl.BlockSpec((R, PC), next_map),
            pl.BlockSpec((R, 1), cent_map),
            const_spec((H, H)), const_spec((H, H)), const_spec((1, H)),
            const_spec((H, H)), const_spec((H, H)), const_spec((1, H)),
        ],
        out_specs=[
            pl.BlockSpec((NUM_GRAPHS, 2 * H), lambda b, g: (0, 0)),
            pl.BlockSpec((NUM_GRAPHS, 2 * H), lambda b, g: (0, 0)),
            pl.BlockSpec((NUM_GRAPHS, 1), lambda b, g: (0, 0)),
        ],
    )
    pmax, psum, cnt = pl.pallas_call(
        _main_body,
        grid_spec=grid_spec,
        out_shape=[
            jax.ShapeDtypeStruct((NUM_GRAPHS, 2 * H), f32),
            jax.ShapeDtypeStruct((NUM_GRAPHS, 2 * H), f32),
            jax.ShapeDtypeStruct((NUM_GRAPHS, 1), f32),
        ],
    )(glohi, hs, hs, hs, pens, pens, pens, batchp2,
      w1a, w1b, b1_, w2a, w2b, b2_)

    wo128 = jnp.pad(Wo.T, ((0, 0), (0, H - 2)))
    bo128 = jnp.pad(bo.reshape(1, 2), ((0, 0), (0, H - 2)))
    out = pl.pallas_call(
        _head_body,
        out_shape=jax.ShapeDtypeStruct((NUM_GRAPHS, 2), f32),
    )(pmax, psum, cnt, Wfl.T, Wfg.T, bfl.reshape(1, H), bfg.reshape(1, H),
      wo128, bo128)
    return out


# speedup vs baseline: 6.7889x; 6.6133x over previous
import functools

import jax
import jax.numpy as jnp
from jax.experimental import pallas as pl
from jax.experimental.pallas import tpu as pltpu
from jax.experimental.pallas import tpu_sc as plsc

N = 50000
NUM_GRAPHS = 256
H = 128
R = 512
NB = 98
NP = NB * R
NP2 = NP + R
C = 144
GW = 128
PC = 19
OFFS = (-4, -3, -2, -1, 1, 2, 3, 4)
NEG = -1e30
KL = N - 1 - ((NB - 1) * R - 4)


def _gather_rows(src, idx):
    rows, cols = idx.shape[0], src.shape[1]
    steps = rows // GW
    idx2 = idx.reshape(1, rows)
    mesh = plsc.VectorSubcoreMesh(core_axis_name="c", subcore_axis_name="s")

    @functools.partial(
        pl.kernel,
        out_type=jax.ShapeDtypeStruct((rows, cols), src.dtype),
        mesh=mesh,
    )
    def gk(x_hbm, i_hbm, o_hbm):
        def body(i_vmem, o_vmem):
            pltpu.sync_copy(x_hbm.at[i_vmem.at[0]], o_vmem)

        pltpu.emit_pipeline(
            body,
            grid=(steps,),
            in_specs=[pl.BlockSpec((1, GW), lambda i: (0, i))],
            out_specs=[pl.BlockSpec((GW, cols), lambda i: (i, 0))],
            core_axis_name=("c", "s"),
            dimension_semantics=(pltpu.PARALLEL,),
        )(i_hbm, o_hbm)

    return gk(src, idx2)


def _glu_body(xg, wlf, wgf, vlin, vgate, bl, bg, hout):
    feat = xg[:, :128]
    tc = xg[:, 128:129]
    lin = jnp.dot(feat, wlf[...], preferred_element_type=jnp.float32) \
        + tc * vlin[...] + bl[...]
    gate = jnp.dot(feat, wgf[...], preferred_element_type=jnp.float32) \
        + tc * vgate[...] + bg[...]
    hout[...] = lin * jax.nn.sigmoid(gate)


def _main_body(glohi, hs_p, hs_c, hs_n, pp, pc, pn, bs_c,
               w1a, w1b, b1, w2a, w2b, b2,
               omax, osum, ocnt):
    b = pl.program_id(0)
    f32 = jnp.float32

    @pl.when(b == 0)
    def _init():
        omax[...] = jnp.full_like(omax, -jnp.inf)
        osum[...] = jnp.zeros_like(osum)
        ocnt[...] = jnp.zeros_like(ocnt)

    h16 = jnp.concatenate([hs_p[R - 8:], hs_c[...], hs_n[:8]], axis=0)
    pst = jnp.concatenate([pp[R - 8:], pc[...], pn[:8]], axis=0)

    M = R + 16
    A1 = jnp.dot(h16[4:M - 4], w1a[...], preferred_element_type=f32) + b1[...]
    B1 = jnp.dot(h16, w1b[...], preferred_element_type=f32)
    p1 = pst[4:M - 4]
    T = None
    for j, d in enumerate(OFFS):
        cand = B1[4 + d:M - 4 + d] + p1[:, j:j + 1]
        T = cand if T is None else jnp.maximum(T, cand)
    x1 = jax.nn.relu(
        A1 + jnp.maximum(T - B1[4:M - 4], p1[:, 18:19]))

    M2 = R + 8
    A2 = jnp.dot(x1[4:M2 - 4], w2a[...], preferred_element_type=f32) + b2[...]
    B2 = jnp.dot(x1, w2b[...], preferred_element_type=f32)
    p2 = pst[8:R + 8]
    T2 = None
    for j, d in enumerate(OFFS):
        cand = B2[4 + d:M2 - 4 + d] + p2[:, 8 + j:9 + j]
        T2 = cand if T2 is None else jnp.maximum(T2, cand)
    T2 = jnp.maximum(T2, B2[4:5] + p2[:, 16:17])
    T2 = jnp.maximum(T2, B2[KL:KL + 1] + p2[:, 17:18])
    x2 = jax.nn.relu(
        A2 + jnp.maximum(T2 - B2[4:M2 - 4], p2[:, 18:19]))

    comb = jnp.concatenate([x1[4:R + 4], x2], axis=1)

    s = b * R
    bsc = bs_c[...]
    growc = s + jax.lax.broadcasted_iota(jnp.int32, (R, 1), 0)
    rowok = growc < N
    glo = glohi[0, b]
    ghi = glohi[1, b]

    def body(gi, carry):
        m = (bsc == gi) & rowok
        mx = jnp.max(jnp.where(m, comb, -jnp.inf), axis=0, keepdims=True)
        sm = jnp.sum(jnp.where(m, comb, 0.0), axis=0, keepdims=True)
        cn = jnp.sum(m.astype(f32), keepdims=True)
        omax[pl.ds(gi, 1), :] = jnp.maximum(omax[pl.ds(gi, 1), :], mx)
        osum[pl.ds(gi, 1), :] = osum[pl.ds(gi, 1), :] + sm
        ocnt[pl.ds(gi, 1), :] = ocnt[pl.ds(gi, 1), :] + cn
        return carry

    jax.lax.fori_loop(glo, ghi + 1, body, 0)


def _head_body(pmax, psum, cnt, wfl, wfg, bfl, bfg, wo, bo, out):
    c = cnt[...]
    maxp = jnp.where(c > 0, pmax[...], 0.0)
    meanp = psum[...] / jnp.maximum(c, 1.0)
    pooled = jnp.concatenate([maxp, meanp], axis=1)
    lin = jnp.dot(pooled, wfl[...], preferred_element_type=jnp.float32) + bfl[...]
    gate = jnp.dot(pooled, wfg[...], preferred_element_type=jnp.float32) + bfg[...]
    hh = lin * jax.nn.sigmoid(gate)
    logits = jnp.dot(hh, wo[...], preferred_element_type=jnp.float32) + bo[...]
    lanes = jax.lax.broadcasted_iota(jnp.int32, logits.shape, 1)
    ok = lanes < 2
    m = jnp.max(jnp.where(ok, logits, -jnp.inf), axis=1, keepdims=True)
    e = jnp.where(ok, jnp.exp(logits - m), 0.0)
    ls = logits - m - jnp.log(jnp.sum(e, axis=1, keepdims=True))
    out[...] = ls[:, 0:2]


def _penalties(batch):
    g = jnp.arange(N, dtype=jnp.int32)
    bpad = jnp.pad(batch, (4, 4), mode="edge")
    cols = []
    pen2 = []
    all_valid = None
    for d in OFFS:
        idx = jnp.clip(g + d, 0, N - 1)
        nb = jax.lax.dynamic_slice(bpad, (4 + d,), (N,))
        vclip = (idx != g) & (nb == batch)
        cols.append(jnp.where(vclip, 0.0, NEG))
        inr = (g + d >= 0) & (g + d <= N - 1)
        pen2.append(jnp.where(vclip & inr, 0.0, NEG))
        all_valid = vclip if all_valid is None else (all_valid & vclip)
    cols += pen2
    x0 = (g >= 1) & (g <= 3) & (batch == batch[0])
    xn = (g >= N - 4) & (g != N - 1) & (batch == batch[N - 1])
    cols.append(jnp.where(x0, 0.0, NEG))
    cols.append(jnp.where(xn, 0.0, NEG))
    cols.append(jnp.where(all_valid, NEG, 0.0))
    pens = jnp.stack(cols, axis=1).astype(jnp.float32)
    pens = jnp.pad(pens, ((R, NP - N), (0, 0)))
    rows2 = jnp.arange(NP2, dtype=jnp.int32)
    inreal = (rows2 >= R) & (rows2 < R + N)
    padrow = jnp.concatenate([jnp.full((PC - 1,), NEG, jnp.float32),
                              jnp.zeros((1,), jnp.float32)])
    return jnp.where(inreal[:, None], pens, padrow)


def kernel(x, batch, Wt, bt, Wl, bl, Wg, bg, W1, b1, W2, b2,
           Wfl, bfl, Wfg, bfg, Wo, bo):
    f32 = jnp.float32
    t = x[:, 0]
    xr = jnp.concatenate([x[:, 1:], x[:, :1]], axis=1)
    xp = jnp.pad(xr, ((0, NP - N), (0, C - x.shape[1])))

    order = jnp.lexsort((t, batch)).astype(jnp.int32)
    order2 = jnp.concatenate([
        jnp.full((R,), order[0], jnp.int32),
        order,
        jnp.full((NP - N,), order[N - 1], jnp.int32),
    ])

    pens = _penalties(batch)

    batchp2 = jnp.pad(batch, (R, NP - N), mode="edge").reshape(NP2, 1)
    blo = batch[jnp.arange(NB, dtype=jnp.int32) * R]
    bhi = batch[jnp.minimum((jnp.arange(NB, dtype=jnp.int32) + 1) * R, N) - 1]
    glohi = jnp.stack([blo, bhi]).astype(jnp.int32)

    wlf = Wl[:, :128].T
    wgf = Wg[:, :128].T
    vlin = (Wl[:, 128:] @ Wt[:, 0]).reshape(1, H)
    vgate = (Wg[:, 128:] @ Wt[:, 0]).reshape(1, H)
    bl_e = (bl + Wl[:, 128:] @ bt).reshape(1, H)
    bg_e = (bg + Wg[:, 128:] @ bt).reshape(1, H)
    w1a, w1b = W1[:, :128].T, W1[:, 128:].T
    w2a, w2b = W2[:, :128].T, W2[:, 128:].T
    b1_, b2_ = b1.reshape(1, H), b2.reshape(1, H)

    csimple = lambda shape: pl.BlockSpec(shape, lambda b: (0, 0))
    h = pl.pallas_call(
        _glu_body,
        grid=(NB,),
        in_specs=[
            pl.BlockSpec((R, C), lambda b: (b, 0)),
            csimple((H, H)), csimple((H, H)),
            csimple((1, H)), csimple((1, H)),
            csimple((1, H)), csimple((1, H)),
        ],
        out_specs=pl.BlockSpec((R, H), lambda b: (b, 0)),
        out_shape=jax.ShapeDtypeStruct((NP, H), f32),
    )(xp, wlf, wgf, vlin, vgate, bl_e, bg_e)

    hs = _gather_rows(h, order2)

    const_spec = lambda shape: pl.BlockSpec(shape, lambda b, g: (0, 0))
    prev_map = lambda b, g: (b, 0)
    cent_map = lambda b, g: (b + 1, 0)
    next_map = lambda b, g: (jnp.minimum(b + 2, NB), 0)

    grid_spec = pltpu.PrefetchScalarGridSpec(
        num_scalar_prefetch=1,
        grid=(NB,),
        in_specs=[
            pl.BlockSpec((R, H), prev_map),
            pl.BlockSpec((R, H), cent_map),
            pl.BlockSpec((R, H), next_map),
            pl.BlockSpec((R, PC), prev_map),
            pl.BlockSpec((R, PC), cent_map),
            pl.BlockSpec((R, PC), next_map),
            pl.BlockSpec((R, 1), cent_map),
            const_spec((H, H)), const_spec((H, H)), const_spec((1, H)),
            const_spec((H, H)), const_spec((H, H)), const_spec((1, H)),
        ],
        out_specs=[
            pl.BlockSpec((NUM_GRAPHS, 2 * H), lambda b, g: (0, 0)),
            pl.BlockSpec((NUM_GRAPHS, 2 * H), lambda b, g: (0, 0)),
            pl.BlockSpec((NUM_GRAPHS, 1), lambda b, g: (0, 0)),
        ],
    )
    pmax, psum, cnt = pl.pallas_call(
        _main_body,
        grid_spec=grid_spec,
        out_shape=[
            jax.ShapeDtypeStruct((NUM_GRAPHS, 2 * H), f32),
            jax.ShapeDtypeStruct((NUM_GRAPHS, 2 * H), f32),
            jax.ShapeDtypeStruct((NUM_GRAPHS, 1), f32),
        ],
    )(glohi, hs, hs, hs, pens, pens, pens, batchp2,
      w1a, w1b, b1_, w2a, w2b, b2_)

    wo128 = jnp.pad(Wo.T, ((0, 0), (0, H - 2)))
    bo128 = jnp.pad(bo.reshape(1, 2), ((0, 0), (0, H - 2)))
    out = pl.pallas_call(
        _head_body,
        out_shape=jax.ShapeDtypeStruct((NUM_GRAPHS, 2), f32),
    )(pmax, psum, cnt, Wfl.T, Wfg.T, bfl.reshape(1, H), bfg.reshape(1, H),
      wo128, bo128)
    return out


# speedup vs baseline: 6.8944x; 1.0155x over previous
import functools

import jax
import jax.numpy as jnp
from jax.experimental import pallas as pl
from jax.experimental.pallas import tpu as pltpu
from jax.experimental.pallas import tpu_sc as plsc

N = 50000
NUM_GRAPHS = 256
H = 128
R = 1024
NB = 49
NP = NB * R
NP2 = NP + R
C = 144
GW = 128
PC = 19
OFFS = (-4, -3, -2, -1, 1, 2, 3, 4)
NEG = -1e30
KL = N - 1 - ((NB - 1) * R - 4)


def _gather_rows(src, idx):
    rows, cols = idx.shape[0], src.shape[1]
    steps = rows // GW
    idx2 = idx.reshape(1, rows)
    mesh = plsc.VectorSubcoreMesh(core_axis_name="c", subcore_axis_name="s")

    @functools.partial(
        pl.kernel,
        out_type=jax.ShapeDtypeStruct((rows, cols), src.dtype),
        mesh=mesh,
    )
    def gk(x_hbm, i_hbm, o_hbm):
        def body(i_vmem, o_vmem):
            pltpu.sync_copy(x_hbm.at[i_vmem.at[0]], o_vmem)

        pltpu.emit_pipeline(
            body,
            grid=(steps,),
            in_specs=[pl.BlockSpec((1, GW), lambda i: (0, i))],
            out_specs=[pl.BlockSpec((GW, cols), lambda i: (i, 0))],
            core_axis_name=("c", "s"),
            dimension_semantics=(pltpu.PARALLEL,),
        )(i_hbm, o_hbm)

    return gk(src, idx2)


def _glu_body(xg, wlf, wgf, vlin, vgate, bl, bg, hout):
    feat = xg[:, :128]
    tc = xg[:, 128:129]
    lin = jnp.dot(feat, wlf[...], preferred_element_type=jnp.float32) \
        + tc * vlin[...] + bl[...]
    gate = jnp.dot(feat, wgf[...], preferred_element_type=jnp.float32) \
        + tc * vgate[...] + bg[...]
    hout[...] = lin * jax.nn.sigmoid(gate)


def _main_body(glohi, hs_p, hs_c, hs_n, pp, pc, pn, bs_c,
               w1a, w1b, b1, w2a, w2b, b2,
               omax, osum, ocnt):
    b = pl.program_id(0)
    f32 = jnp.float32

    @pl.when(b == 0)
    def _init():
        omax[...] = jnp.full_like(omax, -jnp.inf)
        osum[...] = jnp.zeros_like(osum)
        ocnt[...] = jnp.zeros_like(ocnt)

    h16 = jnp.concatenate([hs_p[R - 8:], hs_c[...], hs_n[:8]], axis=0)
    pst = jnp.concatenate([pp[R - 8:], pc[...], pn[:8]], axis=0)

    M = R + 16
    A1 = jnp.dot(h16[4:M - 4], w1a[...], preferred_element_type=f32) + b1[...]
    B1 = jnp.dot(h16, w1b[...], preferred_element_type=f32)
    p1 = pst[4:M - 4]
    T = None
    for j, d in enumerate(OFFS):
        cand = B1[4 + d:M - 4 + d] + p1[:, j:j + 1]
        T = cand if T is None else jnp.maximum(T, cand)
    x1 = jax.nn.relu(
        A1 + jnp.maximum(T - B1[4:M - 4], p1[:, 18:19]))

    M2 = R + 8
    A2 = jnp.dot(x1[4:M2 - 4], w2a[...], preferred_element_type=f32) + b2[...]
    B2 = jnp.dot(x1, w2b[...], preferred_element_type=f32)
    p2 = pst[8:R + 8]
    T2 = None
    for j, d in enumerate(OFFS):
        cand = B2[4 + d:M2 - 4 + d] + p2[:, 8 + j:9 + j]
        T2 = cand if T2 is None else jnp.maximum(T2, cand)
    T2 = jnp.maximum(T2, B2[4:5] + p2[:, 16:17])
    T2 = jnp.maximum(T2, B2[KL:KL + 1] + p2[:, 17:18])
    x2 = jax.nn.relu(
        A2 + jnp.maximum(T2 - B2[4:M2 - 4], p2[:, 18:19]))

    comb = jnp.concatenate([x1[4:R + 4], x2], axis=1)

    s = b * R
    bsc = bs_c[...]
    growc = s + jax.lax.broadcasted_iota(jnp.int32, (R, 1), 0)
    rowok = growc < N
    glo = glohi[0, b]
    ghi = glohi[1, b]

    def body(gi, carry):
        m = (bsc == gi) & rowok
        mx = jnp.max(jnp.where(m, comb, -jnp.inf), axis=0, keepdims=True)
        sm = jnp.sum(jnp.where(m, comb, 0.0), axis=0, keepdims=True)
        cn = jnp.sum(m.astype(f32), keepdims=True)
        omax[pl.ds(gi, 1), :] = jnp.maximum(omax[pl.ds(gi, 1), :], mx)
        osum[pl.ds(gi, 1), :] = osum[pl.ds(gi, 1), :] + sm
        ocnt[pl.ds(gi, 1), :] = ocnt[pl.ds(gi, 1), :] + cn
        return carry

    jax.lax.fori_loop(glo, ghi + 1, body, 0)


def _head_body(pmax, psum, cnt, wfl, wfg, bfl, bfg, wo, bo, out):
    c = cnt[...]
    maxp = jnp.where(c > 0, pmax[...], 0.0)
    meanp = psum[...] / jnp.maximum(c, 1.0)
    pooled = jnp.concatenate([maxp, meanp], axis=1)
    lin = jnp.dot(pooled, wfl[...], preferred_element_type=jnp.float32) + bfl[...]
    gate = jnp.dot(pooled, wfg[...], preferred_element_type=jnp.float32) + bfg[...]
    hh = lin * jax.nn.sigmoid(gate)
    logits = jnp.dot(hh, wo[...], preferred_element_type=jnp.float32) + bo[...]
    lanes = jax.lax.broadcasted_iota(jnp.int32, logits.shape, 1)
    ok = lanes < 2
    m = jnp.max(jnp.where(ok, logits, -jnp.inf), axis=1, keepdims=True)
    e = jnp.where(ok, jnp.exp(logits - m), 0.0)
    ls = logits - m - jnp.log(jnp.sum(e, axis=1, keepdims=True))
    out[...] = ls[:, 0:2]


def _penalties(batch):
    g = jnp.arange(N, dtype=jnp.int32)
    bpad = jnp.pad(batch, (4, 4), mode="edge")
    cols = []
    pen2 = []
    all_valid = None
    for d in OFFS:
        idx = jnp.clip(g + d, 0, N - 1)
        nb = jax.lax.dynamic_slice(bpad, (4 + d,), (N,))
        vclip = (idx != g) & (nb == batch)
        cols.append(jnp.where(vclip, 0.0, NEG))
        inr = (g + d >= 0) & (g + d <= N - 1)
        pen2.append(jnp.where(vclip & inr, 0.0, NEG))
        all_valid = vclip if all_valid is None else (all_valid & vclip)
    cols += pen2
    x0 = (g >= 1) & (g <= 3) & (batch == batch[0])
    xn = (g >= N - 4) & (g != N - 1) & (batch == batch[N - 1])
    cols.append(jnp.where(x0, 0.0, NEG))
    cols.append(jnp.where(xn, 0.0, NEG))
    cols.append(jnp.where(all_valid, NEG, 0.0))
    pens = jnp.stack(cols, axis=1).astype(jnp.float32)
    pens = jnp.pad(pens, ((R, NP - N), (0, 0)))
    rows2 = jnp.arange(NP2, dtype=jnp.int32)
    inreal = (rows2 >= R) & (rows2 < R + N)
    padrow = jnp.concatenate([jnp.full((PC - 1,), NEG, jnp.float32),
                              jnp.zeros((1,), jnp.float32)])
    return jnp.where(inreal[:, None], pens, padrow)


def kernel(x, batch, Wt, bt, Wl, bl, Wg, bg, W1, b1, W2, b2,
           Wfl, bfl, Wfg, bfg, Wo, bo):
    f32 = jnp.float32
    t = x[:, 0]
    xr = jnp.concatenate([x[:, 1:], x[:, :1]], axis=1)
    xp = jnp.pad(xr, ((0, NP - N), (0, C - x.shape[1])))

    order = jnp.lexsort((t, batch)).astype(jnp.int32)
    order2 = jnp.concatenate([
        jnp.full((R,), order[0], jnp.int32),
        order,
        jnp.full((NP - N,), order[N - 1], jnp.int32),
    ])

    pens = _penalties(batch)

    batchp2 = jnp.pad(batch, (R, NP - N), mode="edge").reshape(NP2, 1)
    blo = batch[jnp.arange(NB, dtype=jnp.int32) * R]
    bhi = batch[jnp.minimum((jnp.arange(NB, dtype=jnp.int32) + 1) * R, N) - 1]
    glohi = jnp.stack([blo, bhi]).astype(jnp.int32)

    wlf = Wl[:, :128].T
    wgf = Wg[:, :128].T
    vlin = (Wl[:, 128:] @ Wt[:, 0]).reshape(1, H)
    vgate = (Wg[:, 128:] @ Wt[:, 0]).reshape(1, H)
    bl_e = (bl + Wl[:, 128:] @ bt).reshape(1, H)
    bg_e = (bg + Wg[:, 128:] @ bt).reshape(1, H)
    w1a, w1b = W1[:, :128].T, W1[:, 128:].T
    w2a, w2b = W2[:, :128].T, W2[:, 128:].T
    b1_, b2_ = b1.reshape(1, H), b2.reshape(1, H)

    csimple = lambda shape: pl.BlockSpec(shape, lambda b: (0, 0))
    h = pl.pallas_call(
        _glu_body,
        grid=(NB,),
        in_specs=[
            pl.BlockSpec((R, C), lambda b: (b, 0)),
            csimple((H, H)), csimple((H, H)),
            csimple((1, H)), csimple((1, H)),
            csimple((1, H)), csimple((1, H)),
        ],
        out_specs=pl.BlockSpec((R, H), lambda b: (b, 0)),
        out_shape=jax.ShapeDtypeStruct((NP, H), f32),
    )(xp, wlf, wgf, vlin, vgate, bl_e, bg_e)

    hs = _gather_rows(h, order2)

    const_spec = lambda shape: pl.BlockSpec(shape, lambda b, g: (0, 0))
    prev_map = lambda b, g: (b, 0)
    cent_map = lambda b, g: (b + 1, 0)
    next_map = lambda b, g: (jnp.minimum(b + 2, NB), 0)

    grid_spec = pltpu.PrefetchScalarGridSpec(
        num_scalar_prefetch=1,
        grid=(NB,),
        in_specs=[
            pl.BlockSpec((R, H), prev_map),
            pl.BlockSpec((R, H), cent_map),
            pl.BlockSpec((R, H), next_map),
            pl.BlockSpec((R, PC), prev_map),
            pl.BlockSpec((R, PC), cent_map),
            pl.BlockSpec((R, PC), next_map),
            pl.BlockSpec((R, 1), cent_map),
            const_spec((H, H)), const_spec((H, H)), const_spec((1, H)),
            const_spec((H, H)), const_spec((H, H)), const_spec((1, H)),
        ],
        out_specs=[
            pl.BlockSpec((NUM_GRAPHS, 2 * H), lambda b, g: (0, 0)),
            pl.BlockSpec((NUM_GRAPHS, 2 * H), lambda b, g: (0, 0)),
            pl.BlockSpec((NUM_GRAPHS, 1), lambda b, g: (0, 0)),
        ],
    )
    pmax, psum, cnt = pl.pallas_call(
        _main_body,
        grid_spec=grid_spec,
        out_shape=[
            jax.ShapeDtypeStruct((NUM_GRAPHS, 2 * H), f32),
            jax.ShapeDtypeStruct((NUM_GRAPHS, 2 * H), f32),
            jax.ShapeDtypeStruct((NUM_GRAPHS, 1), f32),
        ],
    )(glohi, hs, hs, hs, pens, pens, pens, batchp2,
      w1a, w1b, b1_, w2a, w2b, b2_)

    wo128 = jnp.pad(Wo.T, ((0, 0), (0, H - 2)))
    bo128 = jnp.pad(bo.reshape(1, 2), ((0, 0), (0, H - 2)))
    out = pl.pallas_call(
        _head_body,
        out_shape=jax.ShapeDtypeStruct((NUM_GRAPHS, 2), f32),
    )(pmax, psum, cnt, Wfl.T, Wfg.T, bfl.reshape(1, H), bfg.reshape(1, H),
      wo128, bo128)
    return out


# speedup vs baseline: 7.1985x; 1.0441x over previous
import functools

import jax
import jax.numpy as jnp
from jax.experimental import pallas as pl
from jax.experimental.pallas import tpu as pltpu
from jax.experimental.pallas import tpu_sc as plsc

N = 50000
NUM_GRAPHS = 256
H = 128
R = 1024
NB = 49
NP = NB * R
NP2 = NP + R
C = 144
GW = 128
PC = 19
OFFS = (-4, -3, -2, -1, 1, 2, 3, 4)
NEG = -1e30
KL = N - 1 - ((NB - 1) * R - 4)


def _gather_rows(src, idx):
    rows, cols = idx.shape[0], src.shape[1]
    steps = rows // GW
    idx2 = idx.reshape(1, rows)
    mesh = plsc.VectorSubcoreMesh(core_axis_name="c", subcore_axis_name="s")

    @functools.partial(
        pl.kernel,
        out_type=jax.ShapeDtypeStruct((rows, cols), src.dtype),
        mesh=mesh,
    )
    def gk(x_hbm, i_hbm, o_hbm):
        def body(i_vmem, o_vmem):
            pltpu.sync_copy(x_hbm.at[i_vmem.at[0]], o_vmem)

        pltpu.emit_pipeline(
            body,
            grid=(steps,),
            in_specs=[pl.BlockSpec((1, GW), lambda i: (0, i))],
            out_specs=[pl.BlockSpec((GW, cols), lambda i: (i, 0))],
            core_axis_name=("c", "s"),
            dimension_semantics=(pltpu.PARALLEL,),
        )(i_hbm, o_hbm)

    return gk(src, idx2)


def _main_body(glohi, xs_p, xs_c, xs_n, ts_p, ts_c, ts_n, pp, pc, pn, bs_c,
               wlf, wgf, vlin, vgate, bl, bg,
               w1a, w1b, b1, w2a, w2b, b2,
               omax, osum, ocnt):
    b = pl.program_id(0)
    f32 = jnp.float32

    @pl.when(b == 0)
    def _init():
        omax[...] = jnp.full_like(omax, -jnp.inf)
        osum[...] = jnp.zeros_like(osum)
        ocnt[...] = jnp.zeros_like(ocnt)

    feat = jnp.concatenate([xs_p[R - 8:], xs_c[...], xs_n[:8]], axis=0)
    tc = jnp.concatenate([ts_p[R - 8:], ts_c[...], ts_n[:8]], axis=0)
    pst = jnp.concatenate([pp[R - 8:], pc[...], pn[:8]], axis=0)

    lin = jnp.dot(feat, wlf[...], preferred_element_type=f32) \
        + tc * vlin[...] + bl[...]
    gate = jnp.dot(feat, wgf[...], preferred_element_type=f32) \
        + tc * vgate[...] + bg[...]
    h16 = lin * jax.nn.sigmoid(gate)

    M = R + 16
    A1 = jnp.dot(h16[4:M - 4], w1a[...], preferred_element_type=f32) + b1[...]
    B1 = jnp.dot(h16, w1b[...], preferred_element_type=f32)
    p1 = pst[4:M - 4]
    T = None
    for j, d in enumerate(OFFS):
        cand = B1[4 + d:M - 4 + d] + p1[:, j:j + 1]
        T = cand if T is None else jnp.maximum(T, cand)
    x1 = jax.nn.relu(
        A1 + jnp.maximum(T - B1[4:M - 4], p1[:, 18:19]))

    M2 = R + 8
    A2 = jnp.dot(x1[4:M2 - 4], w2a[...], preferred_element_type=f32) + b2[...]
    B2 = jnp.dot(x1, w2b[...], preferred_element_type=f32)
    p2 = pst[8:R + 8]
    T2 = None
    for j, d in enumerate(OFFS):
        cand = B2[4 + d:M2 - 4 + d] + p2[:, 8 + j:9 + j]
        T2 = cand if T2 is None else jnp.maximum(T2, cand)
    T2 = jnp.maximum(T2, B2[4:5] + p2[:, 16:17])
    T2 = jnp.maximum(T2, B2[KL:KL + 1] + p2[:, 17:18])
    x2 = jax.nn.relu(
        A2 + jnp.maximum(T2 - B2[4:M2 - 4], p2[:, 18:19]))

    comb = jnp.concatenate([x1[4:R + 4], x2], axis=1)

    s = b * R
    bsc = bs_c[...]
    growc = s + jax.lax.broadcasted_iota(jnp.int32, (R, 1), 0)
    rowok = growc < N
    glo = glohi[0, b]
    ghi = glohi[1, b]

    def body(gi, carry):
        m = (bsc == gi) & rowok
        mx = jnp.max(jnp.where(m, comb, -jnp.inf), axis=0, keepdims=True)
        sm = jnp.sum(jnp.where(m, comb, 0.0), axis=0, keepdims=True)
        cn = jnp.sum(m.astype(f32), keepdims=True)
        omax[pl.ds(gi, 1), :] = jnp.maximum(omax[pl.ds(gi, 1), :], mx)
        osum[pl.ds(gi, 1), :] = osum[pl.ds(gi, 1), :] + sm
        ocnt[pl.ds(gi, 1), :] = ocnt[pl.ds(gi, 1), :] + cn
        return carry

    jax.lax.fori_loop(glo, ghi + 1, body, 0)


def _head_body(pmax, psum, cnt, wfl, wfg, bfl, bfg, wo, bo, out):
    c = cnt[...]
    maxp = jnp.where(c > 0, pmax[...], 0.0)
    meanp = psum[...] / jnp.maximum(c, 1.0)
    pooled = jnp.concatenate([maxp, meanp], axis=1)
    lin = jnp.dot(pooled, wfl[...], preferred_element_type=jnp.float32) + bfl[...]
    gate = jnp.dot(pooled, wfg[...], preferred_element_type=jnp.float32) + bfg[...]
    hh = lin * jax.nn.sigmoid(gate)
    logits = jnp.dot(hh, wo[...], preferred_element_type=jnp.float32) + bo[...]
    lanes = jax.lax.broadcasted_iota(jnp.int32, logits.shape, 1)
    ok = lanes < 2
    m = jnp.max(jnp.where(ok, logits, -jnp.inf), axis=1, keepdims=True)
    e = jnp.where(ok, jnp.exp(logits - m), 0.0)
    ls = logits - m - jnp.log(jnp.sum(e, axis=1, keepdims=True))
    out[...] = ls[:, 0:2]


def _penalties(batch):
    g = jnp.arange(N, dtype=jnp.int32)
    bpad = jnp.pad(batch, (4, 4), mode="edge")
    cols = []
    pen2 = []
    all_valid = None
    for d in OFFS:
        idx = jnp.clip(g + d, 0, N - 1)
        nb = jax.lax.dynamic_slice(bpad, (4 + d,), (N,))
        vclip = (idx != g) & (nb == batch)
        cols.append(jnp.where(vclip, 0.0, NEG))
        inr = (g + d >= 0) & (g + d <= N - 1)
        pen2.append(jnp.where(vclip & inr, 0.0, NEG))
        all_valid = vclip if all_valid is None else (all_valid & vclip)
    cols += pen2
    x0 = (g >= 1) & (g <= 3) & (batch == batch[0])
    xn = (g >= N - 4) & (g != N - 1) & (batch == batch[N - 1])
    cols.append(jnp.where(x0, 0.0, NEG))
    cols.append(jnp.where(xn, 0.0, NEG))
    cols.append(jnp.where(all_valid, NEG, 0.0))
    pens = jnp.stack(cols, axis=1).astype(jnp.float32)
    pens = jnp.pad(pens, ((R, NP - N), (0, 0)))
    rows2 = jnp.arange(NP2, dtype=jnp.int32)
    inreal = (rows2 >= R) & (rows2 < R + N)
    padrow = jnp.concatenate([jnp.full((PC - 1,), NEG, jnp.float32),
                              jnp.zeros((1,), jnp.float32)])
    return jnp.where(inreal[:, None], pens, padrow)


def kernel(x, batch, Wt, bt, Wl, bl, Wg, bg, W1, b1, W2, b2,
           Wfl, bfl, Wfg, bfg, Wo, bo):
    f32 = jnp.float32
    t = x[:, 0]
    feat_src = x[:, 1:]

    _, ts, order = jax.lax.sort(
        (batch, t, jnp.arange(N, dtype=jnp.int32)), num_keys=3)
    order = order.astype(jnp.int32)
    order2 = jnp.concatenate([
        jnp.full((R,), order[0], jnp.int32),
        order,
        jnp.full((NP - N,), order[N - 1], jnp.int32),
    ])
    ts2 = jnp.concatenate([
        jnp.full((R,), ts[0], f32),
        ts,
        jnp.full((NP - N,), ts[N - 1], f32),
    ]).reshape(NP2, 1)

    pens = _penalties(batch)

    batchp2 = jnp.pad(batch, (R, NP - N), mode="edge").reshape(NP2, 1)
    blo = batch[jnp.arange(NB, dtype=jnp.int32) * R]
    bhi = batch[jnp.minimum((jnp.arange(NB, dtype=jnp.int32) + 1) * R, N) - 1]
    glohi = jnp.stack([blo, bhi]).astype(jnp.int32)

    wlf = Wl[:, :128].T
    wgf = Wg[:, :128].T
    vlin = (Wl[:, 128:] @ Wt[:, 0]).reshape(1, H)
    vgate = (Wg[:, 128:] @ Wt[:, 0]).reshape(1, H)
    bl_e = (bl + Wl[:, 128:] @ bt).reshape(1, H)
    bg_e = (bg + Wg[:, 128:] @ bt).reshape(1, H)
    w1a, w1b = W1[:, :128].T, W1[:, 128:].T
    w2a, w2b = W2[:, :128].T, W2[:, 128:].T
    b1_, b2_ = b1.reshape(1, H), b2.reshape(1, H)

    xs = _gather_rows(feat_src, order2)

    const_spec = lambda shape: pl.BlockSpec(shape, lambda b, g: (0, 0))
    prev_map = lambda b, g: (b, 0)
    cent_map = lambda b, g: (b + 1, 0)
    next_map = lambda b, g: (jnp.minimum(b + 2, NB), 0)

    grid_spec = pltpu.PrefetchScalarGridSpec(
        num_scalar_prefetch=1,
        grid=(NB,),
        in_specs=[
            pl.BlockSpec((R, H), prev_map),
            pl.BlockSpec((R, H), cent_map),
            pl.BlockSpec((R, H), next_map),
            pl.BlockSpec((R, 1), prev_map),
            pl.BlockSpec((R, 1), cent_map),
            pl.BlockSpec((R, 1), next_map),
            pl.BlockSpec((R, PC), prev_map),
            pl.BlockSpec((R, PC), cent_map),
            pl.BlockSpec((R, PC), next_map),
            pl.BlockSpec((R, 1), cent_map),
            const_spec((H, H)), const_spec((H, H)),
            const_spec((1, H)), const_spec((1, H)),
            const_spec((1, H)), const_spec((1, H)),
            const_spec((H, H)), const_spec((H, H)), const_spec((1, H)),
            const_spec((H, H)), const_spec((H, H)), const_spec((1, H)),
        ],
        out_specs=[
            pl.BlockSpec((NUM_GRAPHS, 2 * H), lambda b, g: (0, 0)),
            pl.BlockSpec((NUM_GRAPHS, 2 * H), lambda b, g: (0, 0)),
            pl.BlockSpec((NUM_GRAPHS, 1), lambda b, g: (0, 0)),
        ],
    )
    pmax, psum, cnt = pl.pallas_call(
        _main_body,
        grid_spec=grid_spec,
        out_shape=[
            jax.ShapeDtypeStruct((NUM_GRAPHS, 2 * H), f32),
            jax.ShapeDtypeStruct((NUM_GRAPHS, 2 * H), f32),
            jax.ShapeDtypeStruct((NUM_GRAPHS, 1), f32),
        ],
    )(glohi, xs, xs, xs, ts2, ts2, ts2, pens, pens, pens, batchp2,
      wlf, wgf, vlin, vgate, bl_e, bg_e,
      w1a, w1b, b1_, w2a, w2b, b2_)

    wo128 = jnp.pad(Wo.T, ((0, 0), (0, H - 2)))
    bo128 = jnp.pad(bo.reshape(1, 2), ((0, 0), (0, H - 2)))
    out = pl.pallas_call(
        _head_body,
        out_shape=jax.ShapeDtypeStruct((NUM_GRAPHS, 2), f32),
    )(pmax, psum, cnt, Wfl.T, Wfg.T, bfl.reshape(1, H), bfg.reshape(1, H),
      wo128, bo128)
    return out


# speedup vs baseline: 7.2636x; 1.0090x over previous
import functools

import jax
import jax.numpy as jnp
from jax.experimental import pallas as pl
from jax.experimental.pallas import tpu as pltpu
from jax.experimental.pallas import tpu_sc as plsc

N = 50000
NUM_GRAPHS = 256
H = 128
R = 1024
NB = 49
NP = NB * R
NP2 = NP + R
C = 144
GW = 128
PC = 19
OFFS = (-4, -3, -2, -1, 1, 2, 3, 4)
NEG = -1e30
KL = N - 1 - ((NB - 1) * R - 4)


def _gather_rows(src, idx):
    rows, cols = idx.shape[0], src.shape[1]
    steps = rows // GW
    idx2 = idx.reshape(1, rows)
    mesh = plsc.VectorSubcoreMesh(core_axis_name="c", subcore_axis_name="s")

    @functools.partial(
        pl.kernel,
        out_type=jax.ShapeDtypeStruct((rows, cols), src.dtype),
        mesh=mesh,
    )
    def gk(x_hbm, i_hbm, o_hbm):
        def body(i_vmem, o_vmem):
            pltpu.sync_copy(x_hbm.at[i_vmem.at[0]], o_vmem)

        pltpu.emit_pipeline(
            body,
            grid=(steps,),
            in_specs=[pl.BlockSpec((1, GW), lambda i: (0, i))],
            out_specs=[pl.BlockSpec((GW, cols), lambda i: (i, 0))],
            core_axis_name=("c", "s"),
            dimension_semantics=(pltpu.PARALLEL,),
        )(i_hbm, o_hbm)

    return gk(src, idx2)


def _main_body(glohi, xs_p, xs_c, xs_n, ts_p, ts_c, ts_n, pp, pc, pn, bs_c,
               wlf, wgf, vlin, vgate, bl, bg,
               w1a, w1b, b1, w2a, w2b, b2,
               omax, osum, ocnt):
    b = pl.program_id(0)
    f32 = jnp.float32

    @pl.when(b == 0)
    def _init():
        omax[...] = jnp.full_like(omax, -jnp.inf)
        osum[...] = jnp.zeros_like(osum)
        ocnt[...] = jnp.zeros_like(ocnt)

    feat = jnp.concatenate([xs_p[R - 8:], xs_c[...], xs_n[:8]], axis=0)
    tc = jnp.concatenate([ts_p[R - 8:], ts_c[...], ts_n[:8]], axis=0)
    mst = jnp.concatenate([pp[R - 8:], pc[...], pn[:8]], axis=0)

    lin = jnp.dot(feat, wlf[...], preferred_element_type=f32) \
        + tc * vlin[...] + bl[...]
    gate = jnp.dot(feat, wgf[...], preferred_element_type=f32) \
        + tc * vgate[...] + bg[...]
    h16 = lin * jax.nn.sigmoid(gate)

    M = R + 16
    A1 = jnp.dot(h16[4:M - 4], w1a[...], preferred_element_type=f32) + b1[...]
    B1 = jnp.dot(h16, w1b[...], preferred_element_type=f32)
    m1 = mst[4:M - 4]
    T = None
    for j, d in enumerate(OFFS):
        cand = jnp.where((m1 & (1 << j)) != 0, B1[4 + d:M - 4 + d], NEG)
        T = cand if T is None else jnp.maximum(T, cand)
    floor1 = jnp.where((m1 & (1 << 18)) != 0, NEG, 0.0)
    x1 = jax.nn.relu(
        A1 + jnp.maximum(T - B1[4:M - 4], floor1))

    M2 = R + 8
    A2 = jnp.dot(x1[4:M2 - 4], w2a[...], preferred_element_type=f32) + b2[...]
    B2 = jnp.dot(x1, w2b[...], preferred_element_type=f32)
    m2 = mst[8:R + 8]
    T2 = None
    for j, d in enumerate(OFFS):
        cand = jnp.where((m2 & (1 << (8 + j))) != 0, B2[4 + d:M2 - 4 + d], NEG)
        T2 = cand if T2 is None else jnp.maximum(T2, cand)
    T2 = jnp.maximum(T2, jnp.where((m2 & (1 << 16)) != 0, B2[4:5], NEG))
    T2 = jnp.maximum(T2, jnp.where((m2 & (1 << 17)) != 0, B2[KL:KL + 1], NEG))
    floor2 = jnp.where((m2 & (1 << 18)) != 0, NEG, 0.0)
    x2 = jax.nn.relu(
        A2 + jnp.maximum(T2 - B2[4:M2 - 4], floor2))

    comb = jnp.concatenate([x1[4:R + 4], x2], axis=1)

    s = b * R
    bsc = bs_c[...]
    growc = s + jax.lax.broadcasted_iota(jnp.int32, (R, 1), 0)
    rowok = growc < N
    glo = glohi[0, b]
    ghi = glohi[1, b]

    def body(gi, carry):
        m = (bsc == gi) & rowok
        mx = jnp.max(jnp.where(m, comb, -jnp.inf), axis=0, keepdims=True)
        sm = jnp.sum(jnp.where(m, comb, 0.0), axis=0, keepdims=True)
        cn = jnp.sum(m.astype(f32), keepdims=True)
        omax[pl.ds(gi, 1), :] = jnp.maximum(omax[pl.ds(gi, 1), :], mx)
        osum[pl.ds(gi, 1), :] = osum[pl.ds(gi, 1), :] + sm
        ocnt[pl.ds(gi, 1), :] = ocnt[pl.ds(gi, 1), :] + cn
        return carry

    jax.lax.fori_loop(glo, ghi + 1, body, 0)


def _head_body(pmax, psum, cnt, wfl, wfg, bfl, bfg, wo, bo, out):
    c = cnt[...]
    maxp = jnp.where(c > 0, pmax[...], 0.0)
    meanp = psum[...] / jnp.maximum(c, 1.0)
    pooled = jnp.concatenate([maxp, meanp], axis=1)
    lin = jnp.dot(pooled, wfl[...], preferred_element_type=jnp.float32) + bfl[...]
    gate = jnp.dot(pooled, wfg[...], preferred_element_type=jnp.float32) + bfg[...]
    hh = lin * jax.nn.sigmoid(gate)
    logits = jnp.dot(hh, wo[...], preferred_element_type=jnp.float32) + bo[...]
    lanes = jax.lax.broadcasted_iota(jnp.int32, logits.shape, 1)
    ok = lanes < 2
    m = jnp.max(jnp.where(ok, logits, -jnp.inf), axis=1, keepdims=True)
    e = jnp.where(ok, jnp.exp(logits - m), 0.0)
    ls = logits - m - jnp.log(jnp.sum(e, axis=1, keepdims=True))
    out[...] = ls[:, 0:2]


def _penalties(batch):
    g = jnp.arange(N, dtype=jnp.int32)
    bpad = jnp.pad(batch, (4, 4), mode="edge")
    bm = jnp.zeros((N,), jnp.int32)
    all_valid = None
    for j, d in enumerate(OFFS):
        idx = jnp.clip(g + d, 0, N - 1)
        nb = jax.lax.dynamic_slice(bpad, (4 + d,), (N,))
        vclip = (idx != g) & (nb == batch)
        inr = (g + d >= 0) & (g + d <= N - 1)
        bm += vclip.astype(jnp.int32) << j
        bm += (vclip & inr).astype(jnp.int32) << (8 + j)
        all_valid = vclip if all_valid is None else (all_valid & vclip)
    x0 = (g >= 1) & (g <= 3) & (batch == batch[0])
    xn = (g >= N - 4) & (g != N - 1) & (batch == batch[N - 1])
    bm += x0.astype(jnp.int32) << 16
    bm += xn.astype(jnp.int32) << 17
    bm += all_valid.astype(jnp.int32) << 18
    return jnp.pad(bm, (R, NP - N)).reshape(NP2, 1)


def kernel(x, batch, Wt, bt, Wl, bl, Wg, bg, W1, b1, W2, b2,
           Wfl, bfl, Wfg, bfg, Wo, bo):
    f32 = jnp.float32
    t = x[:, 0]
    feat_src = x[:, 1:]

    _, ts, order = jax.lax.sort(
        (batch, t, jnp.arange(N, dtype=jnp.int32)), num_keys=3)
    order = order.astype(jnp.int32)
    order2 = jnp.concatenate([
        jnp.full((R,), order[0], jnp.int32),
        order,
        jnp.full((NP - N,), order[N - 1], jnp.int32),
    ])
    ts2 = jnp.concatenate([
        jnp.full((R,), ts[0], f32),
        ts,
        jnp.full((NP - N,), ts[N - 1], f32),
    ]).reshape(NP2, 1)

    pens = _penalties(batch)

    batchp2 = jnp.pad(batch, (R, NP - N), mode="edge").reshape(NP2, 1)
    blo = batch[jnp.arange(NB, dtype=jnp.int32) * R]
    bhi = batch[jnp.minimum((jnp.arange(NB, dtype=jnp.int32) + 1) * R, N) - 1]
    glohi = jnp.stack([blo, bhi]).astype(jnp.int32)

    wlf = Wl[:, :128].T
    wgf = Wg[:, :128].T
    vlin = (Wl[:, 128:] @ Wt[:, 0]).reshape(1, H)
    vgate = (Wg[:, 128:] @ Wt[:, 0]).reshape(1, H)
    bl_e = (bl + Wl[:, 128:] @ bt).reshape(1, H)
    bg_e = (bg + Wg[:, 128:] @ bt).reshape(1, H)
    w1a, w1b = W1[:, :128].T, W1[:, 128:].T
    w2a, w2b = W2[:, :128].T, W2[:, 128:].T
    b1_, b2_ = b1.reshape(1, H), b2.reshape(1, H)

    xs = _gather_rows(feat_src, order2)

    const_spec = lambda shape: pl.BlockSpec(shape, lambda b, g: (0, 0))
    prev_map = lambda b, g: (b, 0)
    cent_map = lambda b, g: (b + 1, 0)
    next_map = lambda b, g: (jnp.minimum(b + 2, NB), 0)

    grid_spec = pltpu.PrefetchScalarGridSpec(
        num_scalar_prefetch=1,
        grid=(NB,),
        in_specs=[
            pl.BlockSpec((R, H), prev_map),
            pl.BlockSpec((R, H), cent_map),
            pl.BlockSpec((R, H), next_map),
            pl.BlockSpec((R, 1), prev_map),
            pl.BlockSpec((R, 1), cent_map),
            pl.BlockSpec((R, 1), next_map),
            pl.BlockSpec((R, 1), prev_map),
            pl.BlockSpec((R, 1), cent_map),
            pl.BlockSpec((R, 1), next_map),
            pl.BlockSpec((R, 1), cent_map),
            const_spec((H, H)), const_spec((H, H)),
            const_spec((1, H)), const_spec((1, H)),
            const_spec((1, H)), const_spec((1, H)),
            const_spec((H, H)), const_spec((H, H)), const_spec((1, H)),
            const_spec((H, H)), const_spec((H, H)), const_spec((1, H)),
        ],
        out_specs=[
            pl.BlockSpec((NUM_GRAPHS, 2 * H), lambda b, g: (0, 0)),
            pl.BlockSpec((NUM_GRAPHS, 2 * H), lambda b, g: (0, 0)),
            pl.BlockSpec((NUM_GRAPHS, 1), lambda b, g: (0, 0)),
        ],
    )
    pmax, psum, cnt = pl.pallas_call(
        _main_body,
        grid_spec=grid_spec,
        out_shape=[
            jax.ShapeDtypeStruct((NUM_GRAPHS, 2 * H), f32),
            jax.ShapeDtypeStruct((NUM_GRAPHS, 2 * H), f32),
            jax.ShapeDtypeStruct((NUM_GRAPHS, 1), f32),
        ],
    )(glohi, xs, xs, xs, ts2, ts2, ts2, pens, pens, pens, batchp2,
      wlf, wgf, vlin, vgate, bl_e, bg_e,
      w1a, w1b, b1_, w2a, w2b, b2_)

    wo128 = jnp.pad(Wo.T, ((0, 0), (0, H - 2)))
    bo128 = jnp.pad(bo.reshape(1, 2), ((0, 0), (0, H - 2)))
    out = pl.pallas_call(
        _head_body,
        out_shape=jax.ShapeDtypeStruct((NUM_GRAPHS, 2), f32),
    )(pmax, psum, cnt, Wfl.T, Wfg.T, bfl.reshape(1, H), bfg.reshape(1, H),
      wo128, bo128)
    return out
